# Initial kernel scaffold; baseline (speedup 1.0000x reference)
import functools

import jax
import jax.numpy as jnp
from jax import lax
from jax.experimental import pallas as pl
from jax.experimental.pallas import tpu as pltpu
from jax.experimental.pallas import tpu_sc as plsc

_NC = 2
_NS = 16


def _sc_aggregate(x, src, dst, zeros_tile):
    n, d = x.shape
    e = src.shape[0]
    nw = _NC * _NS
    edges_per_tile = e // nw
    ch = 80
    n_chunks = edges_per_tile // ch
    rows_per_tile = n // _NS

    mesh = plsc.VectorSubcoreMesh(core_axis_name="c", subcore_axis_name="s")

    @functools.partial(
        pl.kernel,
        out_type=jax.ShapeDtypeStruct((_NC, n, d), jnp.float32),
        mesh=mesh,
        scratch_types=[
            pltpu.VMEM((ch,), jnp.int32),
            pltpu.VMEM((ch,), jnp.int32),
            pltpu.VMEM((ch, d), jnp.float32),
            pltpu.VMEM((rows_per_tile, d), jnp.float32),
            pltpu.VMEM_SHARED((n, d), jnp.float32),
            pltpu.SemaphoreType.DMA,
        ],
    )
    def k(x_hbm, src_hbm, dst_hbm, z_hbm, out_hbm, sidx, didx, rows, stage,
          acc, sem):
        c = lax.axis_index("c")
        s = lax.axis_index("s")
        wid = c * _NS + s

        pltpu.sync_copy(z_hbm, stage)
        pltpu.sync_copy(stage, acc.at[pl.ds(s * rows_per_tile, rows_per_tile)])
        plsc.subcore_barrier()

        base = wid * edges_per_tile

        def body(i, carry):
            off = base + i * ch
            pltpu.sync_copy(src_hbm.at[pl.ds(off, ch)], sidx)
            pltpu.sync_copy(dst_hbm.at[pl.ds(off, ch)], didx)
            pltpu.async_copy(x_hbm.at[sidx], rows, sem).wait()
            pltpu.sync_copy(rows, acc.at[didx], add=True)
            return carry

        lax.fori_loop(0, n_chunks, body, 0)
        plsc.subcore_barrier()

        r0 = s * rows_per_tile
        pltpu.sync_copy(acc.at[pl.ds(r0, rows_per_tile)], stage)
        pltpu.sync_copy(stage, out_hbm.at[c].at[pl.ds(r0, rows_per_tile)])

    return k(x, src, dst, zeros_tile)


def _mlp_body(x_ref, p0_ref, p1_ref, scale_ref, w1_ref, b1_ref, w2_ref,
              b2_ref, t_ref, sum_ref, sq_ref):
    i = pl.program_id(0)
    h0 = scale_ref[0, 0] * x_ref[...] + p0_ref[...] + p1_ref[...]
    a = jnp.maximum(
        jnp.dot(h0, w1_ref[...], preferred_element_type=jnp.float32)
        + b1_ref[...], 0.0)
    t = jnp.maximum(
        jnp.dot(a, w2_ref[...], preferred_element_type=jnp.float32)
        + b2_ref[...], 0.0)
    t_ref[...] = t
    s = jnp.sum(t, axis=0, keepdims=True)
    s2 = jnp.sum(t * t, axis=0, keepdims=True)

    @pl.when(i == 0)
    def _():
        sum_ref[...] = s
        sq_ref[...] = s2

    @pl.when(i > 0)
    def _():
        sum_ref[...] += s
        sq_ref[...] += s2


def _mlp(x, p0, p1, scale, w1, b1, w2, b2, br):
    n, d = x.shape
    nb = n // br
    row = lambda i: (i, 0)
    rep = lambda i: (0, 0)
    return pl.pallas_call(
        _mlp_body,
        grid=(nb,),
        in_specs=[
            pl.BlockSpec((br, d), row),
            pl.BlockSpec((br, d), row),
            pl.BlockSpec((br, d), row),
            pl.BlockSpec(memory_space=pltpu.SMEM),
            pl.BlockSpec((d, d), rep),
            pl.BlockSpec((1, d), rep),
            pl.BlockSpec((d, d), rep),
            pl.BlockSpec((1, d), rep),
        ],
        out_specs=[
            pl.BlockSpec((br, d), row),
            pl.BlockSpec((1, d), rep),
            pl.BlockSpec((1, d), rep),
        ],
        out_shape=[
            jax.ShapeDtypeStruct((n, d), jnp.float32),
            jax.ShapeDtypeStruct((1, d), jnp.float32),
            jax.ShapeDtypeStruct((1, d), jnp.float32),
        ],
    )(x, p0, p1, scale, w1, b1, w2, b2)


def _bn_apply_body(n_nodes, t_ref, sum_ref, sq_ref, g_ref, be_ref, h_ref):
    mean = sum_ref[...] / n_nodes
    var = sq_ref[...] / n_nodes - mean * mean
    inv = lax.rsqrt(var + 1e-5)
    h = g_ref[...] * (t_ref[...] - mean) * inv + be_ref[...]
    h_ref[...] = jnp.maximum(h, 0.0)


def _bn_apply(t, ssum, ssq, g, be, br):
    n, d = t.shape
    nb = n // br
    row = lambda i: (i, 0)
    rep = lambda i: (0, 0)
    return pl.pallas_call(
        functools.partial(_bn_apply_body, float(n)),
        grid=(nb,),
        in_specs=[
            pl.BlockSpec((br, d), row),
            pl.BlockSpec((1, d), rep),
            pl.BlockSpec((1, d), rep),
            pl.BlockSpec((1, d), rep),
            pl.BlockSpec((1, d), rep),
        ],
        out_specs=pl.BlockSpec((br, d), row),
        out_shape=jax.ShapeDtypeStruct((n, d), jnp.float32),
    )(t, ssum, ssq, g, be)


def _bn_apply_mm_body(n_nodes, t_ref, sum_ref, sq_ref, g_ref, be_ref, wo_ref,
                      bo_ref, o_ref):
    mean = sum_ref[...] / n_nodes
    var = sq_ref[...] / n_nodes - mean * mean
    inv = lax.rsqrt(var + 1e-5)
    h = g_ref[...] * (t_ref[...] - mean) * inv + be_ref[...]
    h = jnp.maximum(h, 0.0)
    o_ref[...] = (
        jnp.dot(h, wo_ref[...], preferred_element_type=jnp.float32)
        + bo_ref[...])


def _bn_apply_mm(t, ssum, ssq, g, be, wo, bo, br):
    n, d = t.shape
    dout = wo.shape[1]
    nb = n // br
    row = lambda i: (i, 0)
    rep = lambda i: (0, 0)
    return pl.pallas_call(
        functools.partial(_bn_apply_mm_body, float(n)),
        grid=(nb,),
        in_specs=[
            pl.BlockSpec((br, d), row),
            pl.BlockSpec((1, d), rep),
            pl.BlockSpec((1, d), rep),
            pl.BlockSpec((1, d), rep),
            pl.BlockSpec((1, d), rep),
            pl.BlockSpec((d, dout), rep),
            pl.BlockSpec((1, dout), rep),
        ],
        out_specs=pl.BlockSpec((br, dout), row),
        out_shape=jax.ShapeDtypeStruct((n, dout), jnp.float32),
    )(t, ssum, ssq, g, be, wo, bo)


@jax.jit
def kernel(x, edge_index, eps1, W11, b11, W12, b12, g1, be1, eps2, W21, b21,
           W22, b22, g2, be2, Wo, bo):
    n, d = x.shape
    src = edge_index[0]
    dst = edge_index[1]
    zeros_tile = jnp.zeros((n // _NS, d), jnp.float32)
    br = 2000

    r2 = lambda v: v.reshape(1, -1)
    scale1 = (1.0 + eps1).reshape(1, 1)
    scale2 = (1.0 + eps2).reshape(1, 1)

    p = _sc_aggregate(x, src, dst, zeros_tile)
    t1, s1, q1 = _mlp(x, p[0], p[1], scale1, W11, r2(b11), W12, r2(b12), br)
    h1 = _bn_apply(t1, s1, q1, r2(g1), r2(be1), br)

    p2 = _sc_aggregate(h1, src, dst, zeros_tile)
    t2, s2, q2 = _mlp(h1, p2[0], p2[1], scale2, W21, r2(b21), W22, r2(b22),
                      br)
    out = _bn_apply_mm(t2, s2, q2, r2(g2), r2(be2), Wo, r2(bo), br)
    return out


# speedup vs baseline: 3.0027x; 3.0027x over previous
import functools

import jax
import jax.numpy as jnp
from jax import lax
from jax.experimental import pallas as pl
from jax.experimental.pallas import tpu as pltpu
from jax.experimental.pallas import tpu_sc as plsc

_NC = 2
_NS = 16


def _sc_aggregate(x, src, dst, zeros_tile):
    n, d = x.shape
    e = src.shape[0]
    nw = _NC * _NS
    edges_per_tile = e // nw
    ch = 80
    n_chunks = edges_per_tile // ch
    n_pad = -(-n // (_NS * 8)) * (_NS * 8)
    rows_per_tile = n_pad // _NS

    mesh = plsc.VectorSubcoreMesh(
        core_axis_name="c", subcore_axis_name="s", num_cores=_NC,
        num_subcores=_NS)

    @functools.partial(
        pl.kernel,
        out_type=jax.ShapeDtypeStruct((_NC, n_pad, d), jnp.float32),
        mesh=mesh,
        scratch_types=[
            pltpu.VMEM((ch,), jnp.int32),
            pltpu.VMEM((ch,), jnp.int32),
            pltpu.VMEM((ch, d), jnp.float32),
            pltpu.VMEM((rows_per_tile, d), jnp.float32),
            pltpu.VMEM_SHARED((n_pad, d), jnp.float32),
            pltpu.SemaphoreType.DMA,
        ],
        compiler_params=pltpu.CompilerParams(use_tc_tiling_on_sc=False),
    )
    def k(x_hbm, src_hbm, dst_hbm, z_hbm, out_hbm, sidx, didx, rows, stage,
          acc, sem):
        c = lax.axis_index("c")
        s = lax.axis_index("s")
        wid = c * _NS + s

        pltpu.sync_copy(z_hbm, stage)
        pltpu.sync_copy(stage, acc.at[pl.ds(s * rows_per_tile, rows_per_tile)])
        plsc.subcore_barrier()

        base = wid * edges_per_tile

        def body(i, carry):
            off = base + i * ch
            pltpu.sync_copy(src_hbm.at[pl.ds(off, ch)], sidx)
            pltpu.sync_copy(dst_hbm.at[pl.ds(off, ch)], didx)
            pltpu.async_copy(x_hbm.at[sidx], rows, sem).wait()
            pltpu.sync_copy(rows, acc.at[didx], add=True)
            return carry

        lax.fori_loop(0, n_chunks, body, 0)
        plsc.subcore_barrier()

        r0 = s * rows_per_tile
        pltpu.sync_copy(acc.at[pl.ds(r0, rows_per_tile)], stage)
        pltpu.sync_copy(stage, out_hbm.at[c].at[pl.ds(r0, rows_per_tile)])

    return k(x, src, dst, zeros_tile)[:, :n, :]


def _mlp_body(x_ref, p0_ref, p1_ref, scale_ref, w1_ref, b1_ref, w2_ref,
              b2_ref, t_ref, sum_ref, sq_ref):
    i = pl.program_id(0)
    h0 = scale_ref[0, 0] * x_ref[...] + p0_ref[...] + p1_ref[...]
    a = jnp.maximum(
        jnp.dot(h0, w1_ref[...], preferred_element_type=jnp.float32)
        + b1_ref[...], 0.0)
    t = jnp.maximum(
        jnp.dot(a, w2_ref[...], preferred_element_type=jnp.float32)
        + b2_ref[...], 0.0)
    t_ref[...] = t
    s = jnp.sum(t, axis=0, keepdims=True)
    s2 = jnp.sum(t * t, axis=0, keepdims=True)

    @pl.when(i == 0)
    def _():
        sum_ref[...] = s
        sq_ref[...] = s2

    @pl.when(i > 0)
    def _():
        sum_ref[...] += s
        sq_ref[...] += s2


def _mlp(x, p0, p1, scale, w1, b1, w2, b2, br):
    n, d = x.shape
    nb = n // br
    row = lambda i: (i, 0)
    rep = lambda i: (0, 0)
    return pl.pallas_call(
        _mlp_body,
        grid=(nb,),
        in_specs=[
            pl.BlockSpec((br, d), row),
            pl.BlockSpec((br, d), row),
            pl.BlockSpec((br, d), row),
            pl.BlockSpec(memory_space=pltpu.SMEM),
            pl.BlockSpec((d, d), rep),
            pl.BlockSpec((1, d), rep),
            pl.BlockSpec((d, d), rep),
            pl.BlockSpec((1, d), rep),
        ],
        out_specs=[
            pl.BlockSpec((br, d), row),
            pl.BlockSpec((1, d), rep),
            pl.BlockSpec((1, d), rep),
        ],
        out_shape=[
            jax.ShapeDtypeStruct((n, d), jnp.float32),
            jax.ShapeDtypeStruct((1, d), jnp.float32),
            jax.ShapeDtypeStruct((1, d), jnp.float32),
        ],
    )(x, p0, p1, scale, w1, b1, w2, b2)


def _bn_apply_body(n_nodes, t_ref, sum_ref, sq_ref, g_ref, be_ref, h_ref):
    mean = sum_ref[...] / n_nodes
    var = sq_ref[...] / n_nodes - mean * mean
    inv = lax.rsqrt(var + 1e-5)
    h = g_ref[...] * (t_ref[...] - mean) * inv + be_ref[...]
    h_ref[...] = jnp.maximum(h, 0.0)


def _bn_apply(t, ssum, ssq, g, be, br):
    n, d = t.shape
    nb = n // br
    row = lambda i: (i, 0)
    rep = lambda i: (0, 0)
    return pl.pallas_call(
        functools.partial(_bn_apply_body, float(n)),
        grid=(nb,),
        in_specs=[
            pl.BlockSpec((br, d), row),
            pl.BlockSpec((1, d), rep),
            pl.BlockSpec((1, d), rep),
            pl.BlockSpec((1, d), rep),
            pl.BlockSpec((1, d), rep),
        ],
        out_specs=pl.BlockSpec((br, d), row),
        out_shape=jax.ShapeDtypeStruct((n, d), jnp.float32),
    )(t, ssum, ssq, g, be)


def _bn_apply_mm_body(n_nodes, t_ref, sum_ref, sq_ref, g_ref, be_ref, wo_ref,
                      bo_ref, o_ref):
    mean = sum_ref[...] / n_nodes
    var = sq_ref[...] / n_nodes - mean * mean
    inv = lax.rsqrt(var + 1e-5)
    h = g_ref[...] * (t_ref[...] - mean) * inv + be_ref[...]
    h = jnp.maximum(h, 0.0)
    o_ref[...] = (
        jnp.dot(h, wo_ref[...], preferred_element_type=jnp.float32)
        + bo_ref[...])


def _bn_apply_mm(t, ssum, ssq, g, be, wo, bo, br):
    n, d = t.shape
    dout = wo.shape[1]
    nb = n // br
    row = lambda i: (i, 0)
    rep = lambda i: (0, 0)
    return pl.pallas_call(
        functools.partial(_bn_apply_mm_body, float(n)),
        grid=(nb,),
        in_specs=[
            pl.BlockSpec((br, d), row),
            pl.BlockSpec((1, d), rep),
            pl.BlockSpec((1, d), rep),
            pl.BlockSpec((1, d), rep),
            pl.BlockSpec((1, d), rep),
            pl.BlockSpec((d, dout), rep),
            pl.BlockSpec((1, dout), rep),
        ],
        out_specs=pl.BlockSpec((br, dout), row),
        out_shape=jax.ShapeDtypeStruct((n, dout), jnp.float32),
    )(t, ssum, ssq, g, be, wo, bo)


@jax.jit
def kernel(x, edge_index, eps1, W11, b11, W12, b12, g1, be1, eps2, W21, b21,
           W22, b22, g2, be2, Wo, bo):
    n, d = x.shape
    src = edge_index[0]
    dst = edge_index[1]
    n_pad = -(-n // (_NS * 8)) * (_NS * 8)
    zeros_tile = jnp.zeros((n_pad // _NS, d // 2), jnp.float32)
    br = 2000

    r2 = lambda v: v.reshape(1, -1)
    scale1 = (1.0 + eps1).reshape(1, 1)
    scale2 = (1.0 + eps2).reshape(1, 1)

    def agg(v):
        vh = v.reshape(n, 2, d // 2)
        pL = _sc_aggregate(vh[:, 0], src, dst, zeros_tile)
        pR = _sc_aggregate(vh[:, 1], src, dst, zeros_tile)
        return jnp.concatenate([pL, pR], axis=2)

    p = agg(x)
    t1, s1, q1 = _mlp(x, p[0], p[1], scale1, W11, r2(b11), W12, r2(b12), br)
    h1 = _bn_apply(t1, s1, q1, r2(g1), r2(be1), br)

    p2 = agg(h1)
    t2, s2, q2 = _mlp(h1, p2[0], p2[1], scale2, W21, r2(b21), W22, r2(b22),
                      br)
    out = _bn_apply_mm(t2, s2, q2, r2(g2), r2(be2), Wo, r2(bo), br)
    return out


# speedup vs baseline: 7.1182x; 2.3706x over previous
import functools

import jax
import jax.numpy as jnp
from jax import lax
from jax.experimental import pallas as pl
from jax.experimental.pallas import tpu as pltpu
from jax.experimental.pallas import tpu_sc as plsc

_NC = 2
_NS = 16


_CH = 125


def _sc_aggregate(x, src2, dst2, zeros_tile):
    n, d = x.shape
    ch = _CH
    total_chunks = src2.shape[0]
    nw = _NC * _NS
    n_chunks = total_chunks // nw
    rows_per_tile = n // _NS

    mesh = plsc.VectorSubcoreMesh(
        core_axis_name="c", subcore_axis_name="s", num_cores=_NC,
        num_subcores=_NS)

    @functools.partial(
        pl.kernel,
        out_type=jax.ShapeDtypeStruct((_NC, n, d), jnp.float32),
        mesh=mesh,
        scratch_types=[
            pltpu.VMEM((n_chunks, ch), jnp.int32),
            pltpu.VMEM((n_chunks, ch), jnp.int32),
            pltpu.VMEM((ch, d), jnp.float32),
            pltpu.VMEM((ch, d), jnp.float32),
            pltpu.VMEM((rows_per_tile, d), jnp.float32),
            pltpu.VMEM_SHARED((n, d), jnp.float32),
            pltpu.SemaphoreType.DMA,
            pltpu.SemaphoreType.DMA,
            pltpu.SemaphoreType.DMA,
            pltpu.SemaphoreType.DMA,
        ],
        compiler_params=pltpu.CompilerParams(use_tc_tiling_on_sc=False),
    )
    def k(x_hbm, src_hbm, dst_hbm, z_hbm, out_hbm, sidx, didx, rows0, rows1,
          stage, acc, gsem0, gsem1, ssem0, ssem1):
        c = lax.axis_index("c")
        s = lax.axis_index("s")
        wid = c * _NS + s

        zcp = pltpu.async_copy(z_hbm, stage, gsem0)
        pltpu.sync_copy(src_hbm.at[pl.ds(wid * n_chunks, n_chunks)], sidx)
        pltpu.sync_copy(dst_hbm.at[pl.ds(wid * n_chunks, n_chunks)], didx)
        zcp.wait()
        pltpu.sync_copy(stage, acc.at[pl.ds(s * rows_per_tile, rows_per_tile)])
        plsc.subcore_barrier()

        def body(j, carry):
            i0 = 2 * j
            i1 = i0 + 1
            g0 = pltpu.async_copy(x_hbm.at[sidx.at[i0]], rows0, gsem0)
            g1 = pltpu.async_copy(x_hbm.at[sidx.at[i1]], rows1, gsem1)
            g0.wait()
            s0 = pltpu.async_copy(rows0, acc.at[didx.at[i0]], ssem0, add=True)
            g1.wait()
            s1 = pltpu.async_copy(rows1, acc.at[didx.at[i1]], ssem1, add=True)
            s0.wait()
            s1.wait()
            return carry

        lax.fori_loop(0, n_chunks // 2, body, 0)
        plsc.subcore_barrier()

        r0 = s * rows_per_tile
        pltpu.sync_copy(acc.at[pl.ds(r0, rows_per_tile)], stage)
        pltpu.sync_copy(stage, out_hbm.at[c].at[pl.ds(r0, rows_per_tile)])

    return k(x, src2, dst2, zeros_tile)


def _mlp_body(x_ref, pl_ref, pr_ref, scale_ref, w1_ref, b1_ref, w2_ref,
              b2_ref, t_ref, sum_ref, sq_ref):
    i = pl.program_id(0)
    agg = jnp.concatenate(
        [pl_ref[0] + pl_ref[1], pr_ref[0] + pr_ref[1]], axis=1)
    h0 = scale_ref[0, 0] * x_ref[...] + agg
    a = jnp.maximum(
        jnp.dot(h0, w1_ref[...], preferred_element_type=jnp.float32)
        + b1_ref[...], 0.0)
    t = jnp.maximum(
        jnp.dot(a, w2_ref[...], preferred_element_type=jnp.float32)
        + b2_ref[...], 0.0)
    t_ref[...] = t
    s = jnp.sum(t, axis=0, keepdims=True)
    s2 = jnp.sum(t * t, axis=0, keepdims=True)

    @pl.when(i == 0)
    def _():
        sum_ref[...] = s
        sq_ref[...] = s2

    @pl.when(i > 0)
    def _():
        sum_ref[...] += s
        sq_ref[...] += s2


def _mlp(x, p_left, p_right, scale, w1, b1, w2, b2, br):
    n, d = x.shape
    nb = n // br
    row = lambda i: (i, 0)
    rep = lambda i: (0, 0)
    return pl.pallas_call(
        _mlp_body,
        grid=(nb,),
        in_specs=[
            pl.BlockSpec((br, d), row),
            pl.BlockSpec((_NC, br, d // 2), lambda i: (0, i, 0)),
            pl.BlockSpec((_NC, br, d // 2), lambda i: (0, i, 0)),
            pl.BlockSpec(memory_space=pltpu.SMEM),
            pl.BlockSpec((d, d), rep),
            pl.BlockSpec((1, d), rep),
            pl.BlockSpec((d, d), rep),
            pl.BlockSpec((1, d), rep),
        ],
        out_specs=[
            pl.BlockSpec((br, d), row),
            pl.BlockSpec((1, d), rep),
            pl.BlockSpec((1, d), rep),
        ],
        out_shape=[
            jax.ShapeDtypeStruct((n, d), jnp.float32),
            jax.ShapeDtypeStruct((1, d), jnp.float32),
            jax.ShapeDtypeStruct((1, d), jnp.float32),
        ],
    )(x, p_left, p_right, scale, w1, b1, w2, b2)


def _bn_apply_body(n_nodes, t_ref, sum_ref, sq_ref, g_ref, be_ref, h_ref):
    mean = sum_ref[...] / n_nodes
    var = sq_ref[...] / n_nodes - mean * mean
    inv = lax.rsqrt(var + 1e-5)
    h = g_ref[...] * (t_ref[...] - mean) * inv + be_ref[...]
    h_ref[...] = jnp.maximum(h, 0.0)


def _bn_apply(t, ssum, ssq, g, be, br):
    n, d = t.shape
    nb = n // br
    row = lambda i: (i, 0)
    rep = lambda i: (0, 0)
    return pl.pallas_call(
        functools.partial(_bn_apply_body, float(n)),
        grid=(nb,),
        in_specs=[
            pl.BlockSpec((br, d), row),
            pl.BlockSpec((1, d), rep),
            pl.BlockSpec((1, d), rep),
            pl.BlockSpec((1, d), rep),
            pl.BlockSpec((1, d), rep),
        ],
        out_specs=pl.BlockSpec((br, d), row),
        out_shape=jax.ShapeDtypeStruct((n, d), jnp.float32),
    )(t, ssum, ssq, g, be)


def _bn_apply_mm_body(n_nodes, t_ref, sum_ref, sq_ref, g_ref, be_ref, wo_ref,
                      bo_ref, o_ref):
    mean = sum_ref[...] / n_nodes
    var = sq_ref[...] / n_nodes - mean * mean
    inv = lax.rsqrt(var + 1e-5)
    h = g_ref[...] * (t_ref[...] - mean) * inv + be_ref[...]
    h = jnp.maximum(h, 0.0)
    o_ref[...] = (
        jnp.dot(h, wo_ref[...], preferred_element_type=jnp.float32)
        + bo_ref[...])


def _bn_apply_mm(t, ssum, ssq, g, be, wo, bo, br):
    n, d = t.shape
    dout = wo.shape[1]
    nb = n // br
    row = lambda i: (i, 0)
    rep = lambda i: (0, 0)
    return pl.pallas_call(
        functools.partial(_bn_apply_mm_body, float(n)),
        grid=(nb,),
        in_specs=[
            pl.BlockSpec((br, d), row),
            pl.BlockSpec((1, d), rep),
            pl.BlockSpec((1, d), rep),
            pl.BlockSpec((1, d), rep),
            pl.BlockSpec((1, d), rep),
            pl.BlockSpec((d, dout), rep),
            pl.BlockSpec((1, dout), rep),
        ],
        out_specs=pl.BlockSpec((br, dout), row),
        out_shape=jax.ShapeDtypeStruct((n, dout), jnp.float32),
    )(t, ssum, ssq, g, be, wo, bo)


@jax.jit
def kernel(x, edge_index, eps1, W11, b11, W12, b12, g1, be1, eps2, W21, b21,
           W22, b22, g2, be2, Wo, bo):
    n, d = x.shape
    src = edge_index[0]
    dst = edge_index[1]
    zeros_tile = jnp.zeros((n // _NS, d // 2), jnp.float32)
    src2 = src.reshape(-1, _CH)
    dst2 = dst.reshape(-1, _CH)
    br = 2000

    r2 = lambda v: v.reshape(1, -1)
    scale1 = (1.0 + eps1).reshape(1, 1)
    scale2 = (1.0 + eps2).reshape(1, 1)

    def agg(v):
        vh = v.reshape(n, 2, d // 2)
        p_left = _sc_aggregate(vh[:, 0], src2, dst2, zeros_tile)
        p_right = _sc_aggregate(vh[:, 1], src2, dst2, zeros_tile)
        return p_left, p_right

    pL, pR = agg(x)
    t1, s1, q1 = _mlp(x, pL, pR, scale1, W11, r2(b11), W12, r2(b12), br)
    h1 = _bn_apply(t1, s1, q1, r2(g1), r2(be1), br)

    pL2, pR2 = agg(h1)
    t2, s2, q2 = _mlp(h1, pL2, pR2, scale2, W21, r2(b21), W22, r2(b22), br)
    out = _bn_apply_mm(t2, s2, q2, r2(g2), r2(be2), Wo, r2(bo), br)
    return out


# speedup vs baseline: 9.3224x; 1.3097x over previous
import functools

import jax
import jax.numpy as jnp
from jax import lax
from jax.experimental import pallas as pl
from jax.experimental.pallas import tpu as pltpu
from jax.experimental.pallas import tpu_sc as plsc

_NC = 2
_NS = 16


_CH = 125


def _sc_aggregate(x, src2, dst2, zeros_tile):
    n, d = x.shape
    ch = _CH
    total_chunks = src2.shape[0]
    nw = _NC * _NS
    n_chunks = total_chunks // nw
    rows_per_tile = n // _NS

    mesh = plsc.VectorSubcoreMesh(
        core_axis_name="c", subcore_axis_name="s", num_cores=_NC,
        num_subcores=_NS)

    nbuf = 4
    n_groups = n_chunks // nbuf

    @functools.partial(
        pl.kernel,
        out_type=jax.ShapeDtypeStruct((_NC, n, d), jnp.float32),
        mesh=mesh,
        scratch_types=[
            pltpu.VMEM((n_chunks, ch), jnp.int32),
            pltpu.VMEM((n_chunks, ch), jnp.int32),
        ] + [pltpu.VMEM((ch, d), jnp.float32) for _ in range(nbuf)] + [
            pltpu.VMEM((_CH, d), jnp.float32),
            pltpu.VMEM_SHARED((n, d), jnp.float32),
        ] + [pltpu.SemaphoreType.DMA for _ in range(2 * nbuf)],
        compiler_params=pltpu.CompilerParams(use_tc_tiling_on_sc=False),
    )
    def k(x_hbm, src_hbm, dst_hbm, z_hbm, out_hbm, sidx, didx, *rest):
        rows = rest[:nbuf]
        stage = rest[nbuf]
        acc = rest[nbuf + 1]
        gsem = rest[nbuf + 2:nbuf + 2 + nbuf]
        ssem = rest[nbuf + 2 + nbuf:]
        c = lax.axis_index("c")
        s = lax.axis_index("s")
        wid = c * _NS + s

        zcp = pltpu.async_copy(z_hbm, stage, gsem[0])
        pltpu.sync_copy(src_hbm.at[pl.ds(wid * n_chunks, n_chunks)], sidx)
        pltpu.sync_copy(dst_hbm.at[pl.ds(wid * n_chunks, n_chunks)], didx)
        zcp.wait()
        for r in range(rows_per_tile // _CH):
            pltpu.sync_copy(
                stage, acc.at[pl.ds(s * rows_per_tile + r * _CH, _CH)])
        plsc.subcore_barrier()

        def gather(i, b):
            return pltpu.async_copy(x_hbm.at[sidx.at[i]], rows[b], gsem[b])

        def gather_wait(i, b):
            pltpu.make_async_copy(x_hbm.at[sidx.at[i]], rows[b],
                                  gsem[b]).wait()

        def scat(i, b):
            return pltpu.async_copy(rows[b], acc.at[didx.at[i]], ssem[b],
                                    add=True)

        def scat_wait(i, b):
            pltpu.make_async_copy(rows[b], acc.at[didx.at[i]], ssem[b]).wait()

        for b in range(nbuf):
            gather(b, b)

        def body(j, carry):
            i0 = nbuf * j
            for b in range(nbuf):
                gather_wait(i0 + b, b)
                scat(i0 + b, b)
            for b in range(nbuf):
                scat_wait(i0 + b, b)
                gather(i0 + nbuf + b, b)
            return carry

        lax.fori_loop(0, n_groups - 1, body, 0)
        ilast = nbuf * (n_groups - 1)
        for b in range(nbuf):
            gather_wait(ilast + b, b)
            scat(ilast + b, b)
        for b in range(nbuf):
            scat_wait(ilast + b, b)
        plsc.subcore_barrier()

        for r in range(rows_per_tile // _CH):
            r0 = s * rows_per_tile + r * _CH
            pltpu.sync_copy(acc.at[pl.ds(r0, _CH)], stage)
            pltpu.sync_copy(stage, out_hbm.at[c].at[pl.ds(r0, _CH)])

    return k(x, src2, dst2, zeros_tile)


def _mlp_body(x_ref, pl_ref, pr_ref, scale_ref, w1_ref, b1_ref, w2_ref,
              b2_ref, t_ref, sum_ref, sq_ref):
    i = pl.program_id(0)
    agg = jnp.concatenate(
        [pl_ref[0] + pl_ref[1], pr_ref[0] + pr_ref[1]], axis=1)
    h0 = scale_ref[0, 0] * x_ref[...] + agg
    a = jnp.maximum(
        jnp.dot(h0, w1_ref[...], preferred_element_type=jnp.float32)
        + b1_ref[...], 0.0)
    t = jnp.maximum(
        jnp.dot(a, w2_ref[...], preferred_element_type=jnp.float32)
        + b2_ref[...], 0.0)
    t_ref[...] = t
    s = jnp.sum(t, axis=0, keepdims=True)
    s2 = jnp.sum(t * t, axis=0, keepdims=True)

    @pl.when(i == 0)
    def _():
        sum_ref[...] = s
        sq_ref[...] = s2

    @pl.when(i > 0)
    def _():
        sum_ref[...] += s
        sq_ref[...] += s2


def _mlp(x, p_left, p_right, scale, w1, b1, w2, b2, br):
    n, d = x.shape
    nb = n // br
    row = lambda i: (i, 0)
    rep = lambda i: (0, 0)
    return pl.pallas_call(
        _mlp_body,
        grid=(nb,),
        in_specs=[
            pl.BlockSpec((br, d), row),
            pl.BlockSpec((_NC, br, d // 2), lambda i: (0, i, 0)),
            pl.BlockSpec((_NC, br, d // 2), lambda i: (0, i, 0)),
            pl.BlockSpec(memory_space=pltpu.SMEM),
            pl.BlockSpec((d, d), rep),
            pl.BlockSpec((1, d), rep),
            pl.BlockSpec((d, d), rep),
            pl.BlockSpec((1, d), rep),
        ],
        out_specs=[
            pl.BlockSpec((br, d), row),
            pl.BlockSpec((1, d), rep),
            pl.BlockSpec((1, d), rep),
        ],
        out_shape=[
            jax.ShapeDtypeStruct((n, d), jnp.float32),
            jax.ShapeDtypeStruct((1, d), jnp.float32),
            jax.ShapeDtypeStruct((1, d), jnp.float32),
        ],
    )(x, p_left, p_right, scale, w1, b1, w2, b2)


def _bn_apply_body(n_nodes, t_ref, sum_ref, sq_ref, g_ref, be_ref, h_ref):
    mean = sum_ref[...] / n_nodes
    var = sq_ref[...] / n_nodes - mean * mean
    inv = lax.rsqrt(var + 1e-5)
    h = g_ref[...] * (t_ref[...] - mean) * inv + be_ref[...]
    h_ref[...] = jnp.maximum(h, 0.0)


def _bn_apply(t, ssum, ssq, g, be, br):
    n, d = t.shape
    nb = n // br
    row = lambda i: (i, 0)
    rep = lambda i: (0, 0)
    return pl.pallas_call(
        functools.partial(_bn_apply_body, float(n)),
        grid=(nb,),
        in_specs=[
            pl.BlockSpec((br, d), row),
            pl.BlockSpec((1, d), rep),
            pl.BlockSpec((1, d), rep),
            pl.BlockSpec((1, d), rep),
            pl.BlockSpec((1, d), rep),
        ],
        out_specs=pl.BlockSpec((br, d), row),
        out_shape=jax.ShapeDtypeStruct((n, d), jnp.float32),
    )(t, ssum, ssq, g, be)


def _bn_apply_mm_body(n_nodes, t_ref, sum_ref, sq_ref, g_ref, be_ref, wo_ref,
                      bo_ref, o_ref):
    mean = sum_ref[...] / n_nodes
    var = sq_ref[...] / n_nodes - mean * mean
    inv = lax.rsqrt(var + 1e-5)
    h = g_ref[...] * (t_ref[...] - mean) * inv + be_ref[...]
    h = jnp.maximum(h, 0.0)
    o_ref[...] = (
        jnp.dot(h, wo_ref[...], preferred_element_type=jnp.float32)
        + bo_ref[...])


def _bn_apply_mm(t, ssum, ssq, g, be, wo, bo, br):
    n, d = t.shape
    dout = wo.shape[1]
    nb = n // br
    row = lambda i: (i, 0)
    rep = lambda i: (0, 0)
    return pl.pallas_call(
        functools.partial(_bn_apply_mm_body, float(n)),
        grid=(nb,),
        in_specs=[
            pl.BlockSpec((br, d), row),
            pl.BlockSpec((1, d), rep),
            pl.BlockSpec((1, d), rep),
            pl.BlockSpec((1, d), rep),
            pl.BlockSpec((1, d), rep),
            pl.BlockSpec((d, dout), rep),
            pl.BlockSpec((1, dout), rep),
        ],
        out_specs=pl.BlockSpec((br, dout), row),
        out_shape=jax.ShapeDtypeStruct((n, dout), jnp.float32),
    )(t, ssum, ssq, g, be, wo, bo)


@jax.jit
def kernel(x, edge_index, eps1, W11, b11, W12, b12, g1, be1, eps2, W21, b21,
           W22, b22, g2, be2, Wo, bo):
    n, d = x.shape
    src = edge_index[0]
    dst = edge_index[1]
    zeros_tile = jnp.zeros((_CH, d // 2), jnp.float32)
    src2 = src.reshape(-1, _CH)
    dst2 = dst.reshape(-1, _CH)
    br = 2000

    r2 = lambda v: v.reshape(1, -1)
    scale1 = (1.0 + eps1).reshape(1, 1)
    scale2 = (1.0 + eps2).reshape(1, 1)

    def agg(v):
        vh = v.reshape(n, 2, d // 2)
        p_left = _sc_aggregate(vh[:, 0], src2, dst2, zeros_tile)
        p_right = _sc_aggregate(vh[:, 1], src2, dst2, zeros_tile)
        return p_left, p_right

    pL, pR = agg(x)
    t1, s1, q1 = _mlp(x, pL, pR, scale1, W11, r2(b11), W12, r2(b12), br)
    h1 = _bn_apply(t1, s1, q1, r2(g1), r2(be1), br)

    pL2, pR2 = agg(h1)
    t2, s2, q2 = _mlp(h1, pL2, pR2, scale2, W21, r2(b21), W22, r2(b22), br)
    out = _bn_apply_mm(t2, s2, q2, r2(g2), r2(be2), Wo, r2(bo), br)
    return out


# speedup vs baseline: 11.1563x; 1.1967x over previous
import functools

import jax
import jax.numpy as jnp
from jax import lax
from jax.experimental import pallas as pl
from jax.experimental.pallas import tpu as pltpu
from jax.experimental.pallas import tpu_sc as plsc

_NC = 2
_NS = 16


_CH = 125


def _sc_aggregate(xview, src_lr, dst2, zeros_tile):
    n2, d = xview.shape
    n = n2 // 2
    ch = _CH
    total_chunks = dst2.shape[0]
    n_chunks = total_chunks // _NS
    rows_per_tile = n // _NS

    mesh = plsc.VectorSubcoreMesh(
        core_axis_name="c", subcore_axis_name="s", num_cores=_NC,
        num_subcores=_NS)

    nbuf = 4
    n_groups = n_chunks // nbuf

    @functools.partial(
        pl.kernel,
        out_type=jax.ShapeDtypeStruct((_NC, n, d), jnp.float32),
        mesh=mesh,
        scratch_types=[
            pltpu.VMEM((n_chunks, ch), jnp.int32),
            pltpu.VMEM((n_chunks, ch), jnp.int32),
        ] + [pltpu.VMEM((ch, d), jnp.float32) for _ in range(nbuf)] + [
            pltpu.VMEM_SHARED((n, d), jnp.float32),
        ] + [pltpu.SemaphoreType.DMA for _ in range(2 * nbuf)],
        compiler_params=pltpu.CompilerParams(use_tc_tiling_on_sc=False),
    )
    def k(x_hbm, src_hbm, dst_hbm, z_hbm, out_hbm, sidx, didx, *rest):
        rows = rest[:nbuf]
        acc = rest[nbuf]
        gsem = rest[nbuf + 1:nbuf + 1 + nbuf]
        ssem = rest[nbuf + 1 + nbuf:]
        c = lax.axis_index("c")
        s = lax.axis_index("s")

        zcp = pltpu.async_copy(
            z_hbm, acc.at[pl.ds(s * rows_per_tile, rows_per_tile)], gsem[0])
        pltpu.sync_copy(src_hbm.at[c].at[pl.ds(s * n_chunks, n_chunks)], sidx)
        pltpu.sync_copy(dst_hbm.at[pl.ds(s * n_chunks, n_chunks)], didx)
        zcp.wait()
        plsc.subcore_barrier()

        def gather(i, b):
            return pltpu.async_copy(x_hbm.at[sidx.at[i]], rows[b], gsem[b])

        def gather_wait(i, b):
            pltpu.make_async_copy(x_hbm.at[sidx.at[i]], rows[b],
                                  gsem[b]).wait()

        def scat(i, b):
            return pltpu.async_copy(rows[b], acc.at[didx.at[i]], ssem[b],
                                    add=True)

        def scat_wait(i, b):
            pltpu.make_async_copy(rows[b], acc.at[didx.at[i]], ssem[b]).wait()

        for b in range(nbuf):
            gather(b, b)

        def body(j, carry):
            i0 = nbuf * j
            for b in range(nbuf):
                gather_wait(i0 + b, b)
                scat(i0 + b, b)
            for b in range(nbuf):
                scat_wait(i0 + b, b)
                gather(i0 + nbuf + b, b)
            return carry

        lax.fori_loop(0, n_groups - 1, body, 0)
        ilast = nbuf * (n_groups - 1)
        for b in range(nbuf):
            gather_wait(ilast + b, b)
            scat(ilast + b, b)
        for b in range(nbuf):
            scat_wait(ilast + b, b)
        plsc.subcore_barrier()

        r0 = s * rows_per_tile
        pltpu.sync_copy(acc.at[pl.ds(r0, rows_per_tile)],
                        out_hbm.at[c].at[pl.ds(r0, rows_per_tile)])

    return k(xview, src_lr, dst2, zeros_tile)


def _mlp_body(x_ref, p_ref, scale_ref, w1_ref, b1_ref, w2_ref,
              b2_ref, t_ref, sum_ref, sq_ref):
    i = pl.program_id(0)
    agg = jnp.concatenate([p_ref[0], p_ref[1]], axis=1)
    h0 = scale_ref[0, 0] * x_ref[...] + agg
    a = jnp.maximum(
        jnp.dot(h0, w1_ref[...], preferred_element_type=jnp.float32)
        + b1_ref[...], 0.0)
    t = jnp.maximum(
        jnp.dot(a, w2_ref[...], preferred_element_type=jnp.float32)
        + b2_ref[...], 0.0)
    t_ref[...] = t
    s = jnp.sum(t, axis=0, keepdims=True)
    s2 = jnp.sum(t * t, axis=0, keepdims=True)

    @pl.when(i == 0)
    def _():
        sum_ref[...] = s
        sq_ref[...] = s2

    @pl.when(i > 0)
    def _():
        sum_ref[...] += s
        sq_ref[...] += s2


def _mlp(x, p, scale, w1, b1, w2, b2, br):
    n, d = x.shape
    nb = n // br
    row = lambda i: (i, 0)
    rep = lambda i: (0, 0)
    return pl.pallas_call(
        _mlp_body,
        grid=(nb,),
        in_specs=[
            pl.BlockSpec((br, d), row),
            pl.BlockSpec((_NC, br, d // 2), lambda i: (0, i, 0)),
            pl.BlockSpec(memory_space=pltpu.SMEM),
            pl.BlockSpec((d, d), rep),
            pl.BlockSpec((1, d), rep),
            pl.BlockSpec((d, d), rep),
            pl.BlockSpec((1, d), rep),
        ],
        out_specs=[
            pl.BlockSpec((br, d), row),
            pl.BlockSpec((1, d), rep),
            pl.BlockSpec((1, d), rep),
        ],
        out_shape=[
            jax.ShapeDtypeStruct((n, d), jnp.float32),
            jax.ShapeDtypeStruct((1, d), jnp.float32),
            jax.ShapeDtypeStruct((1, d), jnp.float32),
        ],
    )(x, p, scale, w1, b1, w2, b2)


def _bn_apply_body(n_nodes, t_ref, sum_ref, sq_ref, g_ref, be_ref, h_ref):
    mean = sum_ref[...] / n_nodes
    var = sq_ref[...] / n_nodes - mean * mean
    inv = lax.rsqrt(var + 1e-5)
    h = g_ref[...] * (t_ref[...] - mean) * inv + be_ref[...]
    h_ref[...] = jnp.maximum(h, 0.0)


def _bn_apply(t, ssum, ssq, g, be, br):
    n, d = t.shape
    nb = n // br
    row = lambda i: (i, 0)
    rep = lambda i: (0, 0)
    return pl.pallas_call(
        functools.partial(_bn_apply_body, float(n)),
        grid=(nb,),
        in_specs=[
            pl.BlockSpec((br, d), row),
            pl.BlockSpec((1, d), rep),
            pl.BlockSpec((1, d), rep),
            pl.BlockSpec((1, d), rep),
            pl.BlockSpec((1, d), rep),
        ],
        out_specs=pl.BlockSpec((br, d), row),
        out_shape=jax.ShapeDtypeStruct((n, d), jnp.float32),
    )(t, ssum, ssq, g, be)


def _bn_apply_mm_body(n_nodes, t_ref, sum_ref, sq_ref, g_ref, be_ref, wo_ref,
                      bo_ref, o_ref):
    mean = sum_ref[...] / n_nodes
    var = sq_ref[...] / n_nodes - mean * mean
    inv = lax.rsqrt(var + 1e-5)
    h = g_ref[...] * (t_ref[...] - mean) * inv + be_ref[...]
    h = jnp.maximum(h, 0.0)
    o_ref[...] = (
        jnp.dot(h, wo_ref[...], preferred_element_type=jnp.float32)
        + bo_ref[...])


def _bn_apply_mm(t, ssum, ssq, g, be, wo, bo, br):
    n, d = t.shape
    dout = wo.shape[1]
    nb = n // br
    row = lambda i: (i, 0)
    rep = lambda i: (0, 0)
    return pl.pallas_call(
        functools.partial(_bn_apply_mm_body, float(n)),
        grid=(nb,),
        in_specs=[
            pl.BlockSpec((br, d), row),
            pl.BlockSpec((1, d), rep),
            pl.BlockSpec((1, d), rep),
            pl.BlockSpec((1, d), rep),
            pl.BlockSpec((1, d), rep),
            pl.BlockSpec((d, dout), rep),
            pl.BlockSpec((1, dout), rep),
        ],
        out_specs=pl.BlockSpec((br, dout), row),
        out_shape=jax.ShapeDtypeStruct((n, dout), jnp.float32),
    )(t, ssum, ssq, g, be, wo, bo)


@jax.jit
def kernel(x, edge_index, eps1, W11, b11, W12, b12, g1, be1, eps2, W21, b21,
           W22, b22, g2, be2, Wo, bo):
    n, d = x.shape
    src = edge_index[0]
    dst = edge_index[1]
    zeros_tile = jnp.zeros((n // _NS, d // 2), jnp.float32)
    src_l = src * 2
    src_lr = jnp.stack(
        [src_l.reshape(-1, _CH), (src_l + 1).reshape(-1, _CH)])
    dst2 = dst.reshape(-1, _CH)
    br = 2000

    r2 = lambda v: v.reshape(1, -1)
    scale1 = (1.0 + eps1).reshape(1, 1)
    scale2 = (1.0 + eps2).reshape(1, 1)

    def agg(v):
        return _sc_aggregate(v.reshape(2 * n, d // 2), src_lr, dst2,
                             zeros_tile)

    p1 = agg(x)
    t1, s1, q1 = _mlp(x, p1, scale1, W11, r2(b11), W12, r2(b12), br)
    h1 = _bn_apply(t1, s1, q1, r2(g1), r2(be1), br)

    p2 = agg(h1)
    t2, s2, q2 = _mlp(h1, p2, scale2, W21, r2(b21), W22, r2(b22), br)
    out = _bn_apply_mm(t2, s2, q2, r2(g2), r2(be2), Wo, r2(bo), br)
    return out


# speedup vs baseline: 11.4880x; 1.0297x over previous
import functools

import jax
import jax.numpy as jnp
from jax import lax
from jax.experimental import pallas as pl
from jax.experimental.pallas import tpu as pltpu
from jax.experimental.pallas import tpu_sc as plsc

_NC = 2
_NS = 16


_CH = 125


def _sc_aggregate(xview, src_lr, dst2, zeros_tile):
    n2, d = xview.shape
    n = n2 // 2
    ch = _CH
    total_chunks = dst2.shape[0]
    n_chunks = total_chunks // _NS
    rows_per_tile = n // _NS

    mesh = plsc.VectorSubcoreMesh(
        core_axis_name="c", subcore_axis_name="s", num_cores=_NC,
        num_subcores=_NS)

    nbuf = 4
    n_groups = n_chunks // nbuf

    @functools.partial(
        pl.kernel,
        out_type=jax.ShapeDtypeStruct((_NC, n, d), jnp.float32),
        mesh=mesh,
        scratch_types=[
            pltpu.VMEM((n_chunks, ch), jnp.int32),
            pltpu.VMEM((n_chunks, ch), jnp.int32),
        ] + [pltpu.VMEM((ch, d), jnp.float32) for _ in range(nbuf)] + [
            pltpu.VMEM_SHARED((n, d), jnp.float32),
        ] + [pltpu.SemaphoreType.DMA for _ in range(2 * nbuf)],
        compiler_params=pltpu.CompilerParams(use_tc_tiling_on_sc=False),
    )
    def k(x_hbm, src_hbm, dst_hbm, z_hbm, out_hbm, sidx, didx, *rest):
        rows = rest[:nbuf]
        acc = rest[nbuf]
        gsem = rest[nbuf + 1:nbuf + 1 + nbuf]
        ssem = rest[nbuf + 1 + nbuf:]
        c = lax.axis_index("c")
        s = lax.axis_index("s")

        zcp = pltpu.async_copy(
            z_hbm, acc.at[pl.ds(s * rows_per_tile, rows_per_tile)], gsem[0])
        pltpu.sync_copy(src_hbm.at[c].at[pl.ds(s * n_chunks, n_chunks)], sidx)
        pltpu.sync_copy(dst_hbm.at[pl.ds(s * n_chunks, n_chunks)], didx)
        zcp.wait()
        plsc.subcore_barrier()

        def gather(i, b):
            return pltpu.async_copy(x_hbm.at[sidx.at[i]], rows[b], gsem[b])

        def gather_wait(i, b):
            pltpu.make_async_copy(x_hbm.at[sidx.at[i]], rows[b],
                                  gsem[b]).wait()

        def scat(i, b):
            return pltpu.async_copy(rows[b], acc.at[didx.at[i]], ssem[b],
                                    add=True)

        def scat_wait(i, b):
            pltpu.make_async_copy(rows[b], acc.at[didx.at[i]], ssem[b]).wait()

        for b in range(nbuf):
            gather(b, b)

        def body(j, carry):
            i0 = nbuf * j
            for b in range(nbuf):
                gather_wait(i0 + b, b)
                scat(i0 + b, b)
            for b in range(nbuf):
                scat_wait(i0 + b, b)
                gather(i0 + nbuf + b, b)
            return carry

        lax.fori_loop(0, n_groups - 1, body, 0)
        ilast = nbuf * (n_groups - 1)
        for b in range(nbuf):
            gather_wait(ilast + b, b)
            scat(ilast + b, b)
        for b in range(nbuf):
            scat_wait(ilast + b, b)
        plsc.subcore_barrier()

        r0 = s * rows_per_tile
        pltpu.sync_copy(acc.at[pl.ds(r0, rows_per_tile)],
                        out_hbm.at[c].at[pl.ds(r0, rows_per_tile)])

    return k(xview, src_lr, dst2, zeros_tile)


def _layer_core(x_ref, p_ref, scale_ref, w1_ref, b1_ref, w2_ref, b2_ref,
                g_ref, be_ref):
    n = x_ref.shape[0]
    agg = jnp.concatenate([p_ref[0], p_ref[1]], axis=1)
    h0 = scale_ref[0, 0] * x_ref[...] + agg
    a = jnp.maximum(
        jnp.dot(h0, w1_ref[...], preferred_element_type=jnp.float32)
        + b1_ref[...], 0.0)
    t = jnp.maximum(
        jnp.dot(a, w2_ref[...], preferred_element_type=jnp.float32)
        + b2_ref[...], 0.0)
    mean = jnp.sum(t, axis=0, keepdims=True) / n
    var = jnp.sum(t * t, axis=0, keepdims=True) / n - mean * mean
    inv = lax.rsqrt(var + 1e-5)
    h = g_ref[...] * (t - mean) * inv + be_ref[...]
    return jnp.maximum(h, 0.0)


def _layer_body(x_ref, p_ref, scale_ref, w1_ref, b1_ref, w2_ref, b2_ref,
                g_ref, be_ref, h_ref):
    h_ref[...] = _layer_core(x_ref, p_ref, scale_ref, w1_ref, b1_ref, w2_ref,
                             b2_ref, g_ref, be_ref)


def _layer_out_body(x_ref, p_ref, scale_ref, w1_ref, b1_ref, w2_ref, b2_ref,
                    g_ref, be_ref, wo_ref, bo_ref, o_ref):
    h = _layer_core(x_ref, p_ref, scale_ref, w1_ref, b1_ref, w2_ref, b2_ref,
                    g_ref, be_ref)
    o_ref[...] = (
        jnp.dot(h, wo_ref[...], preferred_element_type=jnp.float32)
        + bo_ref[...])


def _smem_spec():
    return pl.BlockSpec(memory_space=pltpu.SMEM)


def _layer(x, p, scale, w1, b1, w2, b2, g, be):
    n, d = x.shape
    return pl.pallas_call(
        _layer_body,
        in_specs=[pl.BlockSpec((n, d), lambda: (0, 0)),
                  pl.BlockSpec((_NC, n, d // 2), lambda: (0, 0, 0)),
                  _smem_spec()] + [pl.BlockSpec(b.shape, lambda: (0, 0))
                                   for b in (w1, b1, w2, b2, g, be)],
        out_specs=pl.BlockSpec((n, d), lambda: (0, 0)),
        out_shape=jax.ShapeDtypeStruct((n, d), jnp.float32),
    )(x, p, scale, w1, b1, w2, b2, g, be)


def _layer_out(x, p, scale, w1, b1, w2, b2, g, be, wo, bo):
    n, d = x.shape
    dout = wo.shape[1]
    return pl.pallas_call(
        _layer_out_body,
        in_specs=[pl.BlockSpec((n, d), lambda: (0, 0)),
                  pl.BlockSpec((_NC, n, d // 2), lambda: (0, 0, 0)),
                  _smem_spec()] + [pl.BlockSpec(b.shape, lambda: (0, 0))
                                   for b in (w1, b1, w2, b2, g, be, wo, bo)],
        out_specs=pl.BlockSpec((n, dout), lambda: (0, 0)),
        out_shape=jax.ShapeDtypeStruct((n, dout), jnp.float32),
    )(x, p, scale, w1, b1, w2, b2, g, be, wo, bo)


@jax.jit
def kernel(x, edge_index, eps1, W11, b11, W12, b12, g1, be1, eps2, W21, b21,
           W22, b22, g2, be2, Wo, bo):
    n, d = x.shape
    src = edge_index[0]
    dst = edge_index[1]
    zeros_tile = jnp.zeros((n // _NS, d // 2), jnp.float32)
    src_l = src * 2
    src_lr = jnp.stack(
        [src_l.reshape(-1, _CH), (src_l + 1).reshape(-1, _CH)])
    dst2 = dst.reshape(-1, _CH)

    r2 = lambda v: v.reshape(1, -1)
    scale1 = (1.0 + eps1).reshape(1, 1)
    scale2 = (1.0 + eps2).reshape(1, 1)

    def agg(v):
        return _sc_aggregate(v.reshape(2 * n, d // 2), src_lr, dst2,
                             zeros_tile)

    p1 = agg(x)
    h1 = _layer(x, p1, scale1, W11, r2(b11), W12, r2(b12), r2(g1), r2(be1))
    p2 = agg(h1)
    out = _layer_out(h1, p2, scale2, W21, r2(b21), W22, r2(b22), r2(g2),
                     r2(be2), Wo, r2(bo))
    return out


# speedup vs baseline: 12.3557x; 1.0755x over previous
import functools

import jax
import jax.numpy as jnp
from jax import lax
from jax.experimental import pallas as pl
from jax.experimental.pallas import tpu as pltpu
from jax.experimental.pallas import tpu_sc as plsc

_NC = 2
_NS = 16


_CH = 125


def _sc_aggregate(xview, src_lr, dst2, zeros_tile):
    n2, d = xview.shape
    n = n2 // 2
    ch = _CH
    total_chunks = dst2.shape[0]
    n_chunks = total_chunks // _NS
    rows_per_tile = n // _NS

    mesh = plsc.VectorSubcoreMesh(
        core_axis_name="c", subcore_axis_name="s", num_cores=_NC,
        num_subcores=_NS)

    nbuf = 4
    n_groups = n_chunks // nbuf

    @functools.partial(
        pl.kernel,
        out_type=jax.ShapeDtypeStruct((n, 2 * d), jnp.float32),
        mesh=mesh,
        scratch_types=[
            pltpu.VMEM((n_chunks, ch), jnp.int32),
            pltpu.VMEM((n_chunks, ch), jnp.int32),
        ] + [pltpu.VMEM((ch, d), jnp.float32) for _ in range(nbuf)] + [
            pltpu.VMEM_SHARED((n, d), jnp.float32),
        ] + [pltpu.SemaphoreType.DMA for _ in range(2 * nbuf)],
        compiler_params=pltpu.CompilerParams(use_tc_tiling_on_sc=False),
    )
    def k(x_hbm, src_hbm, dst_hbm, z_hbm, out_hbm, sidx, didx, *rest):
        rows = rest[:nbuf]
        acc = rest[nbuf]
        gsem = rest[nbuf + 1:nbuf + 1 + nbuf]
        ssem = rest[nbuf + 1 + nbuf:]
        c = lax.axis_index("c")
        s = lax.axis_index("s")

        zcp = pltpu.async_copy(
            z_hbm, acc.at[pl.ds(s * rows_per_tile, rows_per_tile)], gsem[0])
        pltpu.sync_copy(src_hbm.at[c].at[pl.ds(s * n_chunks, n_chunks)], sidx)
        pltpu.sync_copy(dst_hbm.at[pl.ds(s * n_chunks, n_chunks)], didx)
        zcp.wait()
        plsc.subcore_barrier()

        def gather(i, b):
            return pltpu.async_copy(x_hbm.at[sidx.at[i]], rows[b], gsem[b])

        def gather_wait(i, b):
            pltpu.make_async_copy(x_hbm.at[sidx.at[i]], rows[b],
                                  gsem[b]).wait()

        def scat(i, b):
            return pltpu.async_copy(rows[b], acc.at[didx.at[i]], ssem[b],
                                    add=True)

        def scat_wait(i, b):
            pltpu.make_async_copy(rows[b], acc.at[didx.at[i]], ssem[b]).wait()

        for b in range(nbuf):
            gather(b, b)

        def body(j, carry):
            i0 = nbuf * j
            for b in range(nbuf):
                gather_wait(i0 + b, b)
                scat(i0 + b, b)
            for b in range(nbuf):
                scat_wait(i0 + b, b)
                gather(i0 + nbuf + b, b)
            return carry

        lax.fori_loop(0, n_groups - 1, body, 0)
        ilast = nbuf * (n_groups - 1)
        for b in range(nbuf):
            gather_wait(ilast + b, b)
            scat(ilast + b, b)
        for b in range(nbuf):
            scat_wait(ilast + b, b)
        plsc.subcore_barrier()

        r0 = s * rows_per_tile
        pltpu.sync_copy(acc.at[pl.ds(r0, rows_per_tile)],
                        out_hbm.at[pl.ds(r0, rows_per_tile), pl.ds(c * d, d)])

    return k(xview, src_lr, dst2, zeros_tile)


def _layer_core(x_ref, p_ref, scale_ref, w1_ref, b1_ref, w2_ref, b2_ref,
                g_ref, be_ref):
    n = x_ref.shape[0]
    h0 = scale_ref[0, 0] * x_ref[...] + p_ref[...]
    a = jnp.maximum(
        jnp.dot(h0, w1_ref[...], preferred_element_type=jnp.float32)
        + b1_ref[...], 0.0)
    t = jnp.maximum(
        jnp.dot(a, w2_ref[...], preferred_element_type=jnp.float32)
        + b2_ref[...], 0.0)
    mean = jnp.sum(t, axis=0, keepdims=True) / n
    var = jnp.sum(t * t, axis=0, keepdims=True) / n - mean * mean
    inv = lax.rsqrt(var + 1e-5)
    h = g_ref[...] * (t - mean) * inv + be_ref[...]
    return jnp.maximum(h, 0.0)


def _layer_body(x_ref, p_ref, scale_ref, w1_ref, b1_ref, w2_ref, b2_ref,
                g_ref, be_ref, h_ref):
    h_ref[...] = _layer_core(x_ref, p_ref, scale_ref, w1_ref, b1_ref, w2_ref,
                             b2_ref, g_ref, be_ref)


def _layer_out_body(x_ref, p_ref, scale_ref, w1_ref, b1_ref, w2_ref, b2_ref,
                    g_ref, be_ref, wo_ref, bo_ref, o_ref):
    h = _layer_core(x_ref, p_ref, scale_ref, w1_ref, b1_ref, w2_ref, b2_ref,
                    g_ref, be_ref)
    o_ref[...] = (
        jnp.dot(h, wo_ref[...], preferred_element_type=jnp.float32)
        + bo_ref[...])


def _smem_spec():
    return pl.BlockSpec(memory_space=pltpu.SMEM)


def _layer(x, p, scale, w1, b1, w2, b2, g, be):
    n, d = x.shape
    return pl.pallas_call(
        _layer_body,
        in_specs=[pl.BlockSpec((n, d), lambda: (0, 0)),
                  pl.BlockSpec((n, d), lambda: (0, 0)),
                  _smem_spec()] + [pl.BlockSpec(b.shape, lambda: (0, 0))
                                   for b in (w1, b1, w2, b2, g, be)],
        out_specs=pl.BlockSpec((n, d), lambda: (0, 0)),
        out_shape=jax.ShapeDtypeStruct((n, d), jnp.float32),
    )(x, p, scale, w1, b1, w2, b2, g, be)


def _layer_out(x, p, scale, w1, b1, w2, b2, g, be, wo, bo):
    n, d = x.shape
    dout = wo.shape[1]
    return pl.pallas_call(
        _layer_out_body,
        in_specs=[pl.BlockSpec((n, d), lambda: (0, 0)),
                  pl.BlockSpec((n, d), lambda: (0, 0)),
                  _smem_spec()] + [pl.BlockSpec(b.shape, lambda: (0, 0))
                                   for b in (w1, b1, w2, b2, g, be, wo, bo)],
        out_specs=pl.BlockSpec((n, dout), lambda: (0, 0)),
        out_shape=jax.ShapeDtypeStruct((n, dout), jnp.float32),
    )(x, p, scale, w1, b1, w2, b2, g, be, wo, bo)


@jax.jit
def kernel(x, edge_index, eps1, W11, b11, W12, b12, g1, be1, eps2, W21, b21,
           W22, b22, g2, be2, Wo, bo):
    n, d = x.shape
    src = edge_index[0]
    dst = edge_index[1]
    zeros_tile = jnp.zeros((n // _NS, d // 2), jnp.float32)
    src_l = src * 2
    src_lr = jnp.stack(
        [src_l.reshape(-1, _CH), (src_l + 1).reshape(-1, _CH)])
    dst2 = dst.reshape(-1, _CH)

    r2 = lambda v: v.reshape(1, -1)
    scale1 = (1.0 + eps1).reshape(1, 1)
    scale2 = (1.0 + eps2).reshape(1, 1)

    def agg(v):
        return _sc_aggregate(v.reshape(2 * n, d // 2), src_lr, dst2,
                             zeros_tile)

    p1 = agg(x)
    h1 = _layer(x, p1, scale1, W11, r2(b11), W12, r2(b12), r2(g1), r2(be1))
    p2 = agg(h1)
    out = _layer_out(h1, p2, scale2, W21, r2(b21), W22, r2(b22), r2(g2),
                     r2(be2), Wo, r2(bo))
    return out


# speedup vs baseline: 12.4991x; 1.0116x over previous
import functools

import jax
import jax.numpy as jnp
from jax import lax
from jax.experimental import pallas as pl
from jax.experimental.pallas import tpu as pltpu
from jax.experimental.pallas import tpu_sc as plsc

_NC = 2
_NS = 16


_CH = 125


def _sc_aggregate(xview, src1, dst1, zeros_tile):
    n2, d = xview.shape
    n = n2 // 2
    ch = _CH
    total_chunks = dst1.shape[0]
    n_chunks = total_chunks // _NS
    rows_per_tile = n // _NS

    mesh = plsc.VectorSubcoreMesh(
        core_axis_name="c", subcore_axis_name="s", num_cores=_NC,
        num_subcores=_NS)

    nbuf = 4
    n_groups = n_chunks // nbuf

    @functools.partial(
        pl.kernel,
        out_type=jax.ShapeDtypeStruct((n, 2 * d), jnp.float32),
        mesh=mesh,
        scratch_types=[
            pltpu.VMEM((n_chunks, ch), jnp.int32),
            pltpu.VMEM((n_chunks, ch), jnp.int32),
        ] + [pltpu.VMEM((ch, d), jnp.float32) for _ in range(nbuf)] + [
            pltpu.VMEM_SHARED((n, d), jnp.float32),
        ] + [pltpu.SemaphoreType.DMA for _ in range(2 * nbuf)],
        compiler_params=pltpu.CompilerParams(use_tc_tiling_on_sc=False),
    )
    def k(x_hbm, src_hbm, dst_hbm, z_hbm, out_hbm, sidx, didx, *rest):
        rows = rest[:nbuf]
        acc = rest[nbuf]
        gsem = rest[nbuf + 1:nbuf + 1 + nbuf]
        ssem = rest[nbuf + 1 + nbuf:]
        c = lax.axis_index("c")
        s = lax.axis_index("s")

        zcp = pltpu.async_copy(
            z_hbm, acc.at[pl.ds(s * rows_per_tile, rows_per_tile)], gsem[0])
        pltpu.sync_copy(src_hbm.at[pl.ds(s * n_chunks, n_chunks)], sidx)
        pltpu.sync_copy(dst_hbm.at[pl.ds(s * n_chunks, n_chunks)], didx)
        zcp.wait()
        plsc.subcore_barrier()
        xs = x_hbm.at[pl.ds(c, n2 - 1)]

        def gather(i, b):
            return pltpu.async_copy(xs.at[sidx.at[i]], rows[b], gsem[b])

        def gather_wait(i, b):
            pltpu.make_async_copy(xs.at[sidx.at[i]], rows[b],
                                  gsem[b]).wait()

        def scat(i, b):
            return pltpu.async_copy(rows[b], acc.at[didx.at[i]], ssem[b],
                                    add=True)

        def scat_wait(i, b):
            pltpu.make_async_copy(rows[b], acc.at[didx.at[i]], ssem[b]).wait()

        for b in range(nbuf):
            gather(b, b)

        def body(j, carry):
            i0 = nbuf * j
            for b in range(nbuf):
                gather_wait(i0 + b, b)
                scat(i0 + b, b)
            for b in range(nbuf):
                scat_wait(i0 + b, b)
                gather(i0 + nbuf + b, b)
            return carry

        lax.fori_loop(0, n_groups - 1, body, 0)
        ilast = nbuf * (n_groups - 1)
        for b in range(nbuf):
            gather_wait(ilast + b, b)
            scat(ilast + b, b)
        for b in range(nbuf):
            scat_wait(ilast + b, b)
        plsc.subcore_barrier()

        r0 = s * rows_per_tile
        pltpu.sync_copy(acc.at[pl.ds(r0, rows_per_tile)],
                        out_hbm.at[pl.ds(r0, rows_per_tile), pl.ds(c * d, d)])

    return k(xview, src1, dst1, zeros_tile)


def _layer_core(x_ref, p_ref, scale_ref, w1_ref, b1_ref, w2_ref, b2_ref,
                g_ref, be_ref):
    n = x_ref.shape[0]
    h0 = scale_ref[0, 0] * x_ref[...] + p_ref[...]
    a = jnp.maximum(
        jnp.dot(h0, w1_ref[...], preferred_element_type=jnp.float32)
        + b1_ref[...], 0.0)
    t = jnp.maximum(
        jnp.dot(a, w2_ref[...], preferred_element_type=jnp.float32)
        + b2_ref[...], 0.0)
    mean = jnp.sum(t, axis=0, keepdims=True) / n
    var = jnp.sum(t * t, axis=0, keepdims=True) / n - mean * mean
    inv = lax.rsqrt(var + 1e-5)
    h = g_ref[...] * (t - mean) * inv + be_ref[...]
    return jnp.maximum(h, 0.0)


def _layer_body(x_ref, p_ref, scale_ref, w1_ref, b1_ref, w2_ref, b2_ref,
                g_ref, be_ref, h_ref):
    h_ref[...] = _layer_core(x_ref, p_ref, scale_ref, w1_ref, b1_ref, w2_ref,
                             b2_ref, g_ref, be_ref)


def _layer_out_body(x_ref, p_ref, scale_ref, w1_ref, b1_ref, w2_ref, b2_ref,
                    g_ref, be_ref, wo_ref, bo_ref, o_ref):
    h = _layer_core(x_ref, p_ref, scale_ref, w1_ref, b1_ref, w2_ref, b2_ref,
                    g_ref, be_ref)
    o_ref[...] = (
        jnp.dot(h, wo_ref[...], preferred_element_type=jnp.float32)
        + bo_ref[...])


def _smem_spec():
    return pl.BlockSpec(memory_space=pltpu.SMEM)


def _layer(x, p, scale, w1, b1, w2, b2, g, be):
    n, d = x.shape
    return pl.pallas_call(
        _layer_body,
        in_specs=[pl.BlockSpec((n, d), lambda: (0, 0)),
                  pl.BlockSpec((n, d), lambda: (0, 0)),
                  _smem_spec()] + [pl.BlockSpec(b.shape, lambda: (0, 0))
                                   for b in (w1, b1, w2, b2, g, be)],
        out_specs=pl.BlockSpec((n, d), lambda: (0, 0)),
        out_shape=jax.ShapeDtypeStruct((n, d), jnp.float32),
    )(x, p, scale, w1, b1, w2, b2, g, be)


def _layer_out(x, p, scale, w1, b1, w2, b2, g, be, wo, bo):
    n, d = x.shape
    dout = wo.shape[1]
    return pl.pallas_call(
        _layer_out_body,
        in_specs=[pl.BlockSpec((n, d), lambda: (0, 0)),
                  pl.BlockSpec((n, d), lambda: (0, 0)),
                  _smem_spec()] + [pl.BlockSpec(b.shape, lambda: (0, 0))
                                   for b in (w1, b1, w2, b2, g, be, wo, bo)],
        out_specs=pl.BlockSpec((n, dout), lambda: (0, 0)),
        out_shape=jax.ShapeDtypeStruct((n, dout), jnp.float32),
    )(x, p, scale, w1, b1, w2, b2, g, be, wo, bo)


@jax.jit
def kernel(x, edge_index, eps1, W11, b11, W12, b12, g1, be1, eps2, W21, b21,
           W22, b22, g2, be2, Wo, bo):
    n, d = x.shape
    src = edge_index[0]
    dst = edge_index[1]
    zeros_tile = jnp.zeros((n // _NS, d // 2), jnp.float32)
    src1 = (src * 2).reshape(-1, _CH)
    dst1 = dst.reshape(-1, _CH)

    r2 = lambda v: v.reshape(1, -1)
    scale1 = (1.0 + eps1).reshape(1, 1)
    scale2 = (1.0 + eps2).reshape(1, 1)

    def agg(v):
        return _sc_aggregate(v.reshape(2 * n, d // 2), src1, dst1,
                             zeros_tile)

    p1 = agg(x)
    h1 = _layer(x, p1, scale1, W11, r2(b11), W12, r2(b12), r2(g1), r2(be1))
    p2 = agg(h1)
    out = _layer_out(h1, p2, scale2, W21, r2(b21), W22, r2(b22), r2(g2),
                     r2(be2), Wo, r2(bo))
    return out


# speedup vs baseline: 12.7303x; 1.0185x over previous
import functools

import jax
import jax.numpy as jnp
from jax import lax
from jax.experimental import pallas as pl
from jax.experimental.pallas import tpu as pltpu
from jax.experimental.pallas import tpu_sc as plsc

_NC = 2
_NS = 16


_CH = 125


def _sc_aggregate(xview, src1, dst1, zeros_tile):
    n2, d = xview.shape
    n = n2 // 2
    ch = _CH
    total_chunks = dst1.shape[0]
    n_chunks = total_chunks // _NS
    rows_per_tile = n // _NS

    mesh = plsc.VectorSubcoreMesh(
        core_axis_name="c", subcore_axis_name="s", num_cores=_NC,
        num_subcores=_NS)

    nbuf = 5
    n_groups = n_chunks // nbuf

    @functools.partial(
        pl.kernel,
        out_type=jax.ShapeDtypeStruct((n, 2 * d), jnp.float32),
        mesh=mesh,
        scratch_types=[
            pltpu.VMEM((n_chunks, ch), jnp.int32),
            pltpu.VMEM((n_chunks, ch), jnp.int32),
        ] + [pltpu.VMEM((ch, d), jnp.float32) for _ in range(nbuf)] + [
            pltpu.VMEM_SHARED((n, d), jnp.float32),
        ] + [pltpu.SemaphoreType.DMA for _ in range(2 * nbuf)],
        compiler_params=pltpu.CompilerParams(use_tc_tiling_on_sc=False),
    )
    def k(x_hbm, src_hbm, dst_hbm, z_hbm, out_hbm, sidx, didx, *rest):
        rows = rest[:nbuf]
        acc = rest[nbuf]
        gsem = rest[nbuf + 1:nbuf + 1 + nbuf]
        ssem = rest[nbuf + 1 + nbuf:]
        c = lax.axis_index("c")
        s = lax.axis_index("s")

        zcp = pltpu.async_copy(
            z_hbm, acc.at[pl.ds(s * rows_per_tile, rows_per_tile)], gsem[0])
        pltpu.sync_copy(src_hbm.at[pl.ds(s * n_chunks, n_chunks)], sidx)
        pltpu.sync_copy(dst_hbm.at[pl.ds(s * n_chunks, n_chunks)], didx)
        zcp.wait()
        plsc.subcore_barrier()
        xs = x_hbm.at[pl.ds(c, n2 - 1)]

        def gather(i, b):
            return pltpu.async_copy(xs.at[sidx.at[i]], rows[b], gsem[b])

        def gather_wait(i, b):
            pltpu.make_async_copy(xs.at[sidx.at[i]], rows[b],
                                  gsem[b]).wait()

        def scat(i, b):
            return pltpu.async_copy(rows[b], acc.at[didx.at[i]], ssem[b],
                                    add=True)

        def scat_wait(i, b):
            pltpu.make_async_copy(rows[b], acc.at[didx.at[i]], ssem[b]).wait()

        for b in range(nbuf):
            gather(b, b)

        def body(j, carry):
            i0 = nbuf * j
            for b in range(nbuf):
                gather_wait(i0 + b, b)
                scat(i0 + b, b)
            for b in range(nbuf):
                scat_wait(i0 + b, b)
                gather(i0 + nbuf + b, b)
            return carry

        lax.fori_loop(0, n_groups - 1, body, 0)
        ilast = nbuf * (n_groups - 1)
        for b in range(nbuf):
            gather_wait(ilast + b, b)
            scat(ilast + b, b)
        for b in range(nbuf):
            scat_wait(ilast + b, b)
        plsc.subcore_barrier()

        r0 = s * rows_per_tile
        pltpu.sync_copy(acc.at[pl.ds(r0, rows_per_tile)],
                        out_hbm.at[pl.ds(r0, rows_per_tile), pl.ds(c * d, d)])

    return k(xview, src1, dst1, zeros_tile)


def _layer_core(x_ref, p_ref, scale_ref, w1_ref, b1_ref, w2_ref, b2_ref,
                g_ref, be_ref):
    n = x_ref.shape[0]
    h0 = scale_ref[0, 0] * x_ref[...] + p_ref[...]
    a = jnp.maximum(
        jnp.dot(h0, w1_ref[...], preferred_element_type=jnp.float32)
        + b1_ref[...], 0.0)
    t = jnp.maximum(
        jnp.dot(a, w2_ref[...], preferred_element_type=jnp.float32)
        + b2_ref[...], 0.0)
    mean = jnp.sum(t, axis=0, keepdims=True) / n
    var = jnp.sum(t * t, axis=0, keepdims=True) / n - mean * mean
    inv = lax.rsqrt(var + 1e-5)
    h = g_ref[...] * (t - mean) * inv + be_ref[...]
    return jnp.maximum(h, 0.0)


def _layer_body(x_ref, p_ref, scale_ref, w1_ref, b1_ref, w2_ref, b2_ref,
                g_ref, be_ref, h_ref):
    h_ref[...] = _layer_core(x_ref, p_ref, scale_ref, w1_ref, b1_ref, w2_ref,
                             b2_ref, g_ref, be_ref)


def _layer_out_body(x_ref, p_ref, scale_ref, w1_ref, b1_ref, w2_ref, b2_ref,
                    g_ref, be_ref, wo_ref, bo_ref, o_ref):
    h = _layer_core(x_ref, p_ref, scale_ref, w1_ref, b1_ref, w2_ref, b2_ref,
                    g_ref, be_ref)
    o_ref[...] = (
        jnp.dot(h, wo_ref[...], preferred_element_type=jnp.float32)
        + bo_ref[...])


def _smem_spec():
    return pl.BlockSpec(memory_space=pltpu.SMEM)


def _layer(x, p, scale, w1, b1, w2, b2, g, be):
    n, d = x.shape
    return pl.pallas_call(
        _layer_body,
        in_specs=[pl.BlockSpec((n, d), lambda: (0, 0)),
                  pl.BlockSpec((n, d), lambda: (0, 0)),
                  _smem_spec()] + [pl.BlockSpec(b.shape, lambda: (0, 0))
                                   for b in (w1, b1, w2, b2, g, be)],
        out_specs=pl.BlockSpec((n, d), lambda: (0, 0)),
        out_shape=jax.ShapeDtypeStruct((n, d), jnp.float32),
    )(x, p, scale, w1, b1, w2, b2, g, be)


def _layer_out(x, p, scale, w1, b1, w2, b2, g, be, wo, bo):
    n, d = x.shape
    dout = wo.shape[1]
    return pl.pallas_call(
        _layer_out_body,
        in_specs=[pl.BlockSpec((n, d), lambda: (0, 0)),
                  pl.BlockSpec((n, d), lambda: (0, 0)),
                  _smem_spec()] + [pl.BlockSpec(b.shape, lambda: (0, 0))
                                   for b in (w1, b1, w2, b2, g, be, wo, bo)],
        out_specs=pl.BlockSpec((n, dout), lambda: (0, 0)),
        out_shape=jax.ShapeDtypeStruct((n, dout), jnp.float32),
    )(x, p, scale, w1, b1, w2, b2, g, be, wo, bo)


@jax.jit
def kernel(x, edge_index, eps1, W11, b11, W12, b12, g1, be1, eps2, W21, b21,
           W22, b22, g2, be2, Wo, bo):
    n, d = x.shape
    src = edge_index[0]
    dst = edge_index[1]
    zeros_tile = jnp.zeros((n // _NS, d // 2), jnp.float32)
    src1 = (src * 2).reshape(-1, _CH)
    dst1 = dst.reshape(-1, _CH)

    r2 = lambda v: v.reshape(1, -1)
    scale1 = (1.0 + eps1).reshape(1, 1)
    scale2 = (1.0 + eps2).reshape(1, 1)

    def agg(v):
        return _sc_aggregate(v.reshape(2 * n, d // 2), src1, dst1,
                             zeros_tile)

    p1 = agg(x)
    h1 = _layer(x, p1, scale1, W11, r2(b11), W12, r2(b12), r2(g1), r2(be1))
    p2 = agg(h1)
    out = _layer_out(h1, p2, scale2, W21, r2(b21), W22, r2(b22), r2(g2),
                     r2(be2), Wo, r2(bo))
    return out


# speedup vs baseline: 13.2862x; 1.0437x over previous
import functools

import jax
import jax.numpy as jnp
from jax import lax
from jax.experimental import pallas as pl
from jax.experimental.pallas import tpu as pltpu
from jax.experimental.pallas import tpu_sc as plsc

_NC = 2
_NS = 16


_CH = 125


def _sc_aggregate(xview, sd, zeros_tile):
    n2, d = xview.shape
    n = n2 // 2
    ch = _CH
    total_chunks = sd.shape[1]
    n_chunks = total_chunks // _NS
    rows_per_tile = n // _NS

    mesh = plsc.VectorSubcoreMesh(
        core_axis_name="c", subcore_axis_name="s", num_cores=_NC,
        num_subcores=_NS)

    nbuf = 5
    n_groups = n_chunks // nbuf

    @functools.partial(
        pl.kernel,
        out_type=jax.ShapeDtypeStruct((n, 2 * d), jnp.float32),
        mesh=mesh,
        scratch_types=[
            pltpu.VMEM((n_chunks, ch), jnp.int32),
            pltpu.VMEM((n_chunks, ch), jnp.int32),
        ] + [pltpu.VMEM((ch, d), jnp.float32) for _ in range(nbuf)] + [
            pltpu.VMEM_SHARED((n, d), jnp.float32),
        ] + [pltpu.SemaphoreType.DMA for _ in range(2 * nbuf)],
        compiler_params=pltpu.CompilerParams(use_tc_tiling_on_sc=False),
    )
    def k(x_hbm, sd_hbm, z_hbm, out_hbm, sidx, didx, *rest):
        rows = rest[:nbuf]
        acc = rest[nbuf]
        gsem = rest[nbuf + 1:nbuf + 1 + nbuf]
        ssem = rest[nbuf + 1 + nbuf:]
        c = lax.axis_index("c")
        s = lax.axis_index("s")

        zcp = pltpu.async_copy(
            z_hbm, acc.at[pl.ds(s * rows_per_tile, rows_per_tile)], gsem[0])
        pltpu.sync_copy(sd_hbm.at[0].at[pl.ds(s * n_chunks, n_chunks)], sidx)
        pltpu.sync_copy(sd_hbm.at[1].at[pl.ds(s * n_chunks, n_chunks)], didx)
        zcp.wait()
        plsc.subcore_barrier()
        xs = x_hbm.at[pl.ds(c, n2 - 1)]

        def gather(i, b):
            return pltpu.async_copy(xs.at[sidx.at[i]], rows[b], gsem[b])

        def gather_wait(i, b):
            pltpu.make_async_copy(xs.at[sidx.at[i]], rows[b],
                                  gsem[b]).wait()

        def scat(i, b):
            return pltpu.async_copy(rows[b], acc.at[didx.at[i]], ssem[b],
                                    add=True)

        def scat_wait(i, b):
            pltpu.make_async_copy(rows[b], acc.at[didx.at[i]], ssem[b]).wait()

        for b in range(nbuf):
            gather(b, b)

        def body(j, carry):
            i0 = nbuf * j
            for b in range(nbuf):
                gather_wait(i0 + b, b)
                scat(i0 + b, b)
            for b in range(nbuf):
                scat_wait(i0 + b, b)
                gather(i0 + nbuf + b, b)
            return carry

        lax.fori_loop(0, n_groups - 1, body, 0)
        ilast = nbuf * (n_groups - 1)
        for b in range(nbuf):
            gather_wait(ilast + b, b)
            scat(ilast + b, b)
        for b in range(nbuf):
            scat_wait(ilast + b, b)
        plsc.subcore_barrier()

        r0 = s * rows_per_tile
        pltpu.sync_copy(acc.at[pl.ds(r0, rows_per_tile)],
                        out_hbm.at[pl.ds(r0, rows_per_tile), pl.ds(c * d, d)])

    return k(xview, sd, zeros_tile)


def _layer_core(x_ref, p_ref, scale_ref, w1_ref, b1_ref, w2_ref, b2_ref,
                g_ref, be_ref):
    n = x_ref.shape[0]
    h0 = scale_ref[0, 0] * x_ref[...] + p_ref[...]
    a = jnp.maximum(
        jnp.dot(h0, w1_ref[...], preferred_element_type=jnp.float32)
        + b1_ref[...], 0.0)
    t = jnp.maximum(
        jnp.dot(a, w2_ref[...], preferred_element_type=jnp.float32)
        + b2_ref[...], 0.0)
    mean = jnp.sum(t, axis=0, keepdims=True) / n
    var = jnp.sum(t * t, axis=0, keepdims=True) / n - mean * mean
    inv = lax.rsqrt(var + 1e-5)
    h = g_ref[...] * (t - mean) * inv + be_ref[...]
    return jnp.maximum(h, 0.0)


def _layer_body(x_ref, p_ref, scale_ref, w1_ref, b1_ref, w2_ref, b2_ref,
                g_ref, be_ref, h_ref):
    h_ref[...] = _layer_core(x_ref, p_ref, scale_ref, w1_ref, b1_ref, w2_ref,
                             b2_ref, g_ref, be_ref)


def _layer_out_body(x_ref, p_ref, scale_ref, w1_ref, b1_ref, w2_ref, b2_ref,
                    g_ref, be_ref, wo_ref, bo_ref, o_ref):
    h = _layer_core(x_ref, p_ref, scale_ref, w1_ref, b1_ref, w2_ref, b2_ref,
                    g_ref, be_ref)
    o_ref[...] = (
        jnp.dot(h, wo_ref[...], preferred_element_type=jnp.float32)
        + bo_ref[...])


def _smem_spec():
    return pl.BlockSpec(memory_space=pltpu.SMEM)


def _layer(x, p, scale, w1, b1, w2, b2, g, be):
    n, d = x.shape
    return pl.pallas_call(
        _layer_body,
        in_specs=[pl.BlockSpec((n, d), lambda: (0, 0)),
                  pl.BlockSpec((n, d), lambda: (0, 0)),
                  _smem_spec()] + [pl.BlockSpec(b.shape, lambda: (0, 0))
                                   for b in (w1, b1, w2, b2, g, be)],
        out_specs=pl.BlockSpec((n, d), lambda: (0, 0)),
        out_shape=jax.ShapeDtypeStruct((n, d), jnp.float32),
    )(x, p, scale, w1, b1, w2, b2, g, be)


def _layer_out(x, p, scale, w1, b1, w2, b2, g, be, wo, bo):
    n, d = x.shape
    dout = wo.shape[1]
    return pl.pallas_call(
        _layer_out_body,
        in_specs=[pl.BlockSpec((n, d), lambda: (0, 0)),
                  pl.BlockSpec((n, d), lambda: (0, 0)),
                  _smem_spec()] + [pl.BlockSpec(b.shape, lambda: (0, 0))
                                   for b in (w1, b1, w2, b2, g, be, wo, bo)],
        out_specs=pl.BlockSpec((n, dout), lambda: (0, 0)),
        out_shape=jax.ShapeDtypeStruct((n, dout), jnp.float32),
    )(x, p, scale, w1, b1, w2, b2, g, be, wo, bo)


@jax.jit
def kernel(x, edge_index, eps1, W11, b11, W12, b12, g1, be1, eps2, W21, b21,
           W22, b22, g2, be2, Wo, bo):
    n, d = x.shape
    zeros_tile = jnp.zeros((n // _NS, d // 2), jnp.float32)
    sd = (edge_index * jnp.array([[2], [1]], jnp.int32)).reshape(2, -1, _CH)

    r2 = lambda v: v.reshape(1, -1)
    scale1 = (1.0 + eps1).reshape(1, 1)
    scale2 = (1.0 + eps2).reshape(1, 1)

    def agg(v):
        return _sc_aggregate(v.reshape(2 * n, d // 2), sd, zeros_tile)

    p1 = agg(x)
    h1 = _layer(x, p1, scale1, W11, r2(b11), W12, r2(b12), r2(g1), r2(be1))
    p2 = agg(h1)
    out = _layer_out(h1, p2, scale2, W21, r2(b21), W22, r2(b22), r2(g2),
                     r2(be2), Wo, r2(bo))
    return out


# speedup vs baseline: 13.4288x; 1.0107x over previous
import functools

import jax
import jax.numpy as jnp
from jax import lax
from jax.experimental import pallas as pl
from jax.experimental.pallas import tpu as pltpu
from jax.experimental.pallas import tpu_sc as plsc

_NC = 2
_NS = 16


_CH = 125


def _sc_aggregate(xview, sd, zeros_tile):
    n2, d = xview.shape
    n = n2 // 2
    ch = _CH
    total_chunks = sd.shape[1]
    n_chunks = total_chunks // _NS
    rows_per_tile = n // _NS

    mesh = plsc.VectorSubcoreMesh(
        core_axis_name="c", subcore_axis_name="s", num_cores=_NC,
        num_subcores=_NS)

    nbuf = 5
    n_groups = n_chunks // nbuf

    @functools.partial(
        pl.kernel,
        out_type=jax.ShapeDtypeStruct((n, 2 * d), jnp.float32),
        mesh=mesh,
        scratch_types=[
            pltpu.VMEM((n_chunks, ch), jnp.int32),
            pltpu.VMEM((n_chunks, ch), jnp.int32),
        ] + [pltpu.VMEM((ch, d), jnp.float32) for _ in range(nbuf)] + [
            pltpu.VMEM_SHARED((n, d), jnp.float32),
        ] + [pltpu.SemaphoreType.DMA for _ in range(2 * nbuf)],
        compiler_params=pltpu.CompilerParams(use_tc_tiling_on_sc=False),
    )
    def k(x_hbm, sd_hbm, z_hbm, out_hbm, sidx, didx, *rest):
        rows = rest[:nbuf]
        acc = rest[nbuf]
        gsem = rest[nbuf + 1:nbuf + 1 + nbuf]
        ssem = rest[nbuf + 1 + nbuf:]
        c = lax.axis_index("c")
        s = lax.axis_index("s")

        zcp = pltpu.async_copy(z_hbm, rows[0], gsem[0])
        pltpu.sync_copy(sd_hbm.at[0].at[pl.ds(s * n_chunks, n_chunks)], sidx)
        zcp.wait()
        zs = []
        for r in range(rows_per_tile // ch):
            zs.append(pltpu.async_copy(
                rows[0], acc.at[pl.ds(s * rows_per_tile + r * ch, ch)],
                ssem[r % nbuf]))
        pltpu.sync_copy(sd_hbm.at[1].at[pl.ds(s * n_chunks, n_chunks)], didx)
        for z in zs:
            z.wait()
        plsc.subcore_barrier()
        xs = x_hbm.at[pl.ds(c, n2 - 1)]

        def gather(i, b):
            return pltpu.async_copy(xs.at[sidx.at[i]], rows[b], gsem[b])

        def gather_wait(i, b):
            pltpu.make_async_copy(xs.at[sidx.at[i]], rows[b],
                                  gsem[b]).wait()

        def scat(i, b):
            return pltpu.async_copy(rows[b], acc.at[didx.at[i]], ssem[b],
                                    add=True)

        def scat_wait(i, b):
            pltpu.make_async_copy(rows[b], acc.at[didx.at[i]], ssem[b]).wait()

        for b in range(nbuf):
            gather(b, b)

        def body(j, carry):
            i0 = nbuf * j
            for b in range(nbuf):
                gather_wait(i0 + b, b)
                scat(i0 + b, b)
            for b in range(nbuf):
                scat_wait(i0 + b, b)
                gather(i0 + nbuf + b, b)
            return carry

        lax.fori_loop(0, n_groups - 1, body, 0)
        ilast = nbuf * (n_groups - 1)
        for b in range(nbuf):
            gather_wait(ilast + b, b)
            scat(ilast + b, b)
        for b in range(nbuf):
            scat_wait(ilast + b, b)
        plsc.subcore_barrier()

        r0 = s * rows_per_tile
        pltpu.sync_copy(acc.at[pl.ds(r0, rows_per_tile)],
                        out_hbm.at[pl.ds(r0, rows_per_tile), pl.ds(c * d, d)])

    return k(xview, sd, zeros_tile)


def _layer_core(x_ref, p_ref, scale_ref, w1_ref, b1_ref, w2_ref, b2_ref,
                g_ref, be_ref):
    n = x_ref.shape[0]
    h0 = scale_ref[0, 0] * x_ref[...] + p_ref[...]
    a = jnp.maximum(
        jnp.dot(h0, w1_ref[...], preferred_element_type=jnp.float32)
        + b1_ref[...], 0.0)
    t = jnp.maximum(
        jnp.dot(a, w2_ref[...], preferred_element_type=jnp.float32)
        + b2_ref[...], 0.0)
    mean = jnp.sum(t, axis=0, keepdims=True) / n
    var = jnp.sum(t * t, axis=0, keepdims=True) / n - mean * mean
    inv = lax.rsqrt(var + 1e-5)
    h = g_ref[...] * (t - mean) * inv + be_ref[...]
    return jnp.maximum(h, 0.0)


def _layer_body(x_ref, p_ref, scale_ref, w1_ref, b1_ref, w2_ref, b2_ref,
                g_ref, be_ref, h_ref):
    h_ref[...] = _layer_core(x_ref, p_ref, scale_ref, w1_ref, b1_ref, w2_ref,
                             b2_ref, g_ref, be_ref)


def _layer_out_body(x_ref, p_ref, scale_ref, w1_ref, b1_ref, w2_ref, b2_ref,
                    g_ref, be_ref, wo_ref, bo_ref, o_ref):
    h = _layer_core(x_ref, p_ref, scale_ref, w1_ref, b1_ref, w2_ref, b2_ref,
                    g_ref, be_ref)
    o_ref[...] = (
        jnp.dot(h, wo_ref[...], preferred_element_type=jnp.float32)
        + bo_ref[...])


def _smem_spec():
    return pl.BlockSpec(memory_space=pltpu.SMEM)


def _layer(x, p, scale, w1, b1, w2, b2, g, be):
    n, d = x.shape
    return pl.pallas_call(
        _layer_body,
        in_specs=[pl.BlockSpec((n, d), lambda: (0, 0)),
                  pl.BlockSpec((n, d), lambda: (0, 0)),
                  _smem_spec()] + [pl.BlockSpec(b.shape, lambda: (0, 0))
                                   for b in (w1, b1, w2, b2, g, be)],
        out_specs=pl.BlockSpec((n, d), lambda: (0, 0)),
        out_shape=jax.ShapeDtypeStruct((n, d), jnp.float32),
    )(x, p, scale, w1, b1, w2, b2, g, be)


def _layer_out(x, p, scale, w1, b1, w2, b2, g, be, wo, bo):
    n, d = x.shape
    dout = wo.shape[1]
    return pl.pallas_call(
        _layer_out_body,
        in_specs=[pl.BlockSpec((n, d), lambda: (0, 0)),
                  pl.BlockSpec((n, d), lambda: (0, 0)),
                  _smem_spec()] + [pl.BlockSpec(b.shape, lambda: (0, 0))
                                   for b in (w1, b1, w2, b2, g, be, wo, bo)],
        out_specs=pl.BlockSpec((n, dout), lambda: (0, 0)),
        out_shape=jax.ShapeDtypeStruct((n, dout), jnp.float32),
    )(x, p, scale, w1, b1, w2, b2, g, be, wo, bo)


@jax.jit
def kernel(x, edge_index, eps1, W11, b11, W12, b12, g1, be1, eps2, W21, b21,
           W22, b22, g2, be2, Wo, bo):
    n, d = x.shape
    zeros_tile = jnp.zeros((_CH, d // 2), jnp.float32)
    sd = (edge_index * jnp.array([[2], [1]], jnp.int32)).reshape(2, -1, _CH)

    r2 = lambda v: v.reshape(1, -1)
    scale1 = (1.0 + eps1).reshape(1, 1)
    scale2 = (1.0 + eps2).reshape(1, 1)

    def agg(v):
        return _sc_aggregate(v.reshape(2 * n, d // 2), sd, zeros_tile)

    p1 = agg(x)
    h1 = _layer(x, p1, scale1, W11, r2(b11), W12, r2(b12), r2(g1), r2(be1))
    p2 = agg(h1)
    out = _layer_out(h1, p2, scale2, W21, r2(b21), W22, r2(b22), r2(g2),
                     r2(be2), Wo, r2(bo))
    return out
